# baseline (speedup 1.0000x reference)
import functools

import jax
import jax.numpy as jnp
from jax import lax
from jax.experimental import pallas as pl
from jax.experimental.pallas import tpu as pltpu
from jax.experimental.pallas import tpu_sc as plsc

D_MODEL = 128
CHUNK = 320
NBUF = 2


@functools.lru_cache(maxsize=None)
def _build_gather(n_idx: int):
    info = plsc.get_sparse_core_info()
    nc, ns = info.num_cores, info.num_subcores
    nw = nc * ns
    assert n_idx % (nw * CHUNK * NBUF) == 0
    b_per_w = n_idx // nw
    n_groups = b_per_w // (CHUNK * NBUF)

    mesh = plsc.VectorSubcoreMesh(core_axis_name="c", subcore_axis_name="s")

    @functools.partial(
        pl.kernel,
        mesh=mesh,
        out_type=jax.ShapeDtypeStruct((n_idx, D_MODEL), jnp.float32),
        scratch_types=(
            [pltpu.VMEM((b_per_w,), jnp.int32)]
            + [pltpu.VMEM((CHUNK, D_MODEL), jnp.float32) for _ in range(NBUF)]
            + [pltpu.SemaphoreType.DMA for _ in range(2 * NBUF)]
        ),
    )
    def gather(t_hbm, table_hbm, out_hbm, idx_v, *bufs_and_sems):
        rows = bufs_and_sems[:NBUF]
        gsem = bufs_and_sems[NBUF:2 * NBUF]
        osem = bufs_and_sems[2 * NBUF:]

        wid = lax.axis_index("s") * nc + lax.axis_index("c")
        base = wid * b_per_w

        pltpu.sync_copy(t_hbm.at[pl.ds(base, b_per_w)], idx_v)

        def start_gather(c, b):
            pltpu.async_copy(
                table_hbm.at[idx_v.at[pl.ds(c * CHUNK, CHUNK)]], rows[b], gsem[b])

        def wait_gather(b):
            pltpu.make_async_copy(
                out_hbm.at[pl.ds(0, CHUNK)], rows[b], gsem[b]).wait()

        def start_out(c, b):
            pltpu.async_copy(
                rows[b], out_hbm.at[pl.ds(base + c * CHUNK, CHUNK)], osem[b])

        def wait_out(c, b):
            pltpu.make_async_copy(
                rows[b], out_hbm.at[pl.ds(base + c * CHUNK, CHUNK)], osem[b]).wait()

        for b in range(NBUF):
            start_gather(b, b)

        def group_body(g, carry):
            c0 = g * NBUF
            for b in range(NBUF):
                wait_gather(b)
                start_out(c0 + b, b)
            for b in range(NBUF):
                wait_out(c0 + b, b)
                start_gather(c0 + NBUF + b, b)
            return carry

        lax.fori_loop(0, n_groups - 1, group_body, 0)

        c0 = (n_groups - 1) * NBUF
        for b in range(NBUF):
            wait_gather(b)
            start_out(c0 + b, b)
        for b in range(NBUF):
            wait_out(c0 + b, b)

    return gather


def kernel(t, pos_embedding):
    b, h = t.shape
    n_idx = b * h
    t_flat = t.reshape(n_idx).astype(jnp.int32)
    out = _build_gather(n_idx)(t_flat, pos_embedding)
    return out.reshape(b, h, D_MODEL)


# speedup vs baseline: 1.6767x; 1.6767x over previous
import functools

import jax
import jax.numpy as jnp
from jax import lax
from jax.experimental import pallas as pl
from jax.experimental.pallas import tpu as pltpu
from jax.experimental.pallas import tpu_sc as plsc

D_MODEL = 128
GRP = 8


@functools.lru_cache(maxsize=None)
def _build_gather(batch: int, hist: int):
    info = plsc.get_sparse_core_info()
    nc, ns = info.num_cores, info.num_subcores
    nw = nc * ns
    assert batch % (nw * GRP) == 0
    b_per_w = batch // nw
    n_groups = b_per_w // GRP

    mesh = plsc.VectorSubcoreMesh(core_axis_name="c", subcore_axis_name="s")

    @functools.partial(
        pl.kernel,
        mesh=mesh,
        out_type=jax.ShapeDtypeStruct((batch, hist, D_MODEL), jnp.float32),
        scratch_types=[
            pltpu.VMEM((b_per_w, hist), jnp.int32),
            pltpu.VMEM((GRP, hist, D_MODEL), jnp.float32),
            pltpu.SemaphoreType.DMA,
        ],
        compiler_params=pltpu.CompilerParams(use_tc_tiling_on_sc=True),
    )
    def gather(t_hbm, table_hbm, out_hbm, idx_v, rows_v, sem):
        wid = lax.axis_index("s") * nc + lax.axis_index("c")
        b0 = wid * b_per_w

        pltpu.sync_copy(t_hbm.at[pl.ds(b0, b_per_w)], idx_v)

        def group_body(g, carry):
            for j in range(GRP):
                pltpu.async_copy(
                    table_hbm.at[idx_v.at[g * GRP + j]], rows_v.at[j], sem)
            for j in range(GRP):
                pltpu.make_async_copy(
                    table_hbm.at[idx_v.at[g * GRP + j]], rows_v.at[j], sem).wait()
            for j in range(GRP):
                pltpu.sync_copy(rows_v.at[j], out_hbm.at[b0 + g * GRP + j])
            return carry

        lax.fori_loop(0, n_groups, group_body, 0)

    return gather


def kernel(t, pos_embedding):
    b, h = t.shape
    out = _build_gather(b, h)(t.astype(jnp.int32), pos_embedding)
    return out


# speedup vs baseline: 2.6964x; 1.6081x over previous
import functools

import jax
import jax.numpy as jnp
from jax import lax
from jax.experimental import pallas as pl
from jax.experimental.pallas import tpu as pltpu
from jax.experimental.pallas import tpu_sc as plsc

D_MODEL = 128
CHUNK = 320
NBUF = 2


@functools.lru_cache(maxsize=None)
def _build_gather(n_idx: int):
    info = plsc.get_sparse_core_info()
    nc, ns = info.num_cores, info.num_subcores
    nw = nc * ns
    assert n_idx % (nw * CHUNK * NBUF) == 0
    b_per_w = n_idx // nw
    n_groups = b_per_w // (CHUNK * NBUF)

    mesh = plsc.VectorSubcoreMesh(core_axis_name="c", subcore_axis_name="s")

    @functools.partial(
        pl.kernel,
        mesh=mesh,
        out_type=jax.ShapeDtypeStruct((n_idx, D_MODEL), jnp.float32),
        scratch_types=(
            [pltpu.VMEM((b_per_w,), jnp.int32)]
            + [pltpu.VMEM((CHUNK, D_MODEL), jnp.float32) for _ in range(NBUF)]
            + [pltpu.SemaphoreType.DMA for _ in range(2 * NBUF)]
        ),
        compiler_params=pltpu.CompilerParams(use_tc_tiling_on_sc=True),
    )
    def gather(t_hbm, table_hbm, out_hbm, idx_v, *bufs_and_sems):
        rows = bufs_and_sems[:NBUF]
        gsem = bufs_and_sems[NBUF:2 * NBUF]
        osem = bufs_and_sems[2 * NBUF:]

        wid = lax.axis_index("s") * nc + lax.axis_index("c")
        base = wid * b_per_w

        pltpu.sync_copy(t_hbm.at[pl.ds(base, b_per_w)], idx_v)

        def start_gather(c, b):
            pltpu.async_copy(
                table_hbm.at[idx_v.at[pl.ds(c * CHUNK, CHUNK)]], rows[b], gsem[b])

        def wait_gather(b):
            pltpu.make_async_copy(
                out_hbm.at[pl.ds(0, CHUNK)], rows[b], gsem[b]).wait()

        def start_out(c, b):
            pltpu.async_copy(
                rows[b], out_hbm.at[pl.ds(base + c * CHUNK, CHUNK)], osem[b])

        def wait_out(c, b):
            pltpu.make_async_copy(
                rows[b], out_hbm.at[pl.ds(base + c * CHUNK, CHUNK)], osem[b]).wait()

        for b in range(NBUF):
            start_gather(b, b)

        def group_body(g, carry):
            c0 = g * NBUF
            for b in range(NBUF):
                wait_gather(b)
                start_out(c0 + b, b)
            for b in range(NBUF):
                wait_out(c0 + b, b)
                start_gather(c0 + NBUF + b, b)
            return carry

        lax.fori_loop(0, n_groups - 1, group_body, 0)

        c0 = (n_groups - 1) * NBUF
        for b in range(NBUF):
            wait_gather(b)
            start_out(c0 + b, b)
        for b in range(NBUF):
            wait_out(c0 + b, b)

    return gather


def kernel(t, pos_embedding):
    b, h = t.shape
    n_idx = b * h
    t_hmaj = jnp.swapaxes(t, 0, 1).reshape(n_idx).astype(jnp.int32)
    out = _build_gather(n_idx)(t_hmaj, pos_embedding)
    return jnp.swapaxes(out.reshape(h, b, D_MODEL), 0, 1)


# speedup vs baseline: 4.6456x; 1.7229x over previous
import functools

import jax
import jax.numpy as jnp
from jax import lax
from jax.experimental import pallas as pl
from jax.experimental.pallas import tpu as pltpu
from jax.experimental.pallas import tpu_sc as plsc

D_MODEL = 128
CHUNK = 320
NBUF = 2


@functools.lru_cache(maxsize=None)
def _build_gather(n_idx: int, n_emb: int):
    info = plsc.get_sparse_core_info()
    nc, ns = info.num_cores, info.num_subcores
    nw = nc * ns
    assert n_idx % (nw * CHUNK * NBUF) == 0
    b_per_w = n_idx // nw
    n_groups = b_per_w // (CHUNK * NBUF)

    mesh = plsc.VectorSubcoreMesh(core_axis_name="c", subcore_axis_name="s")

    @functools.partial(
        pl.kernel,
        mesh=mesh,
        out_type=jax.ShapeDtypeStruct((n_idx, D_MODEL), jnp.float32),
        scratch_types=(
            [pltpu.VMEM((b_per_w,), jnp.int32)]
            + [pltpu.VMEM_SHARED((n_emb, D_MODEL), jnp.float32)]
            + [pltpu.VMEM((CHUNK, D_MODEL), jnp.float32) for _ in range(NBUF)]
            + [pltpu.SemaphoreType.DMA for _ in range(2 * NBUF)]
        ),
        compiler_params=pltpu.CompilerParams(use_tc_tiling_on_sc=True),
    )
    def gather(t_hbm, table_hbm, out_hbm, idx_v, table_sh, *bufs_and_sems):
        rows = bufs_and_sems[:NBUF]
        gsem = bufs_and_sems[NBUF:2 * NBUF]
        osem = bufs_and_sems[2 * NBUF:]

        wid = lax.axis_index("s") * nc + lax.axis_index("c")
        base = wid * b_per_w

        @pl.when(lax.axis_index("s") == 0)
        def _():
            pltpu.sync_copy(table_hbm, table_sh)

        pltpu.sync_copy(t_hbm.at[pl.ds(base, b_per_w)], idx_v)
        plsc.subcore_barrier()

        def start_gather(c, b):
            pltpu.async_copy(
                table_sh.at[idx_v.at[pl.ds(c * CHUNK, CHUNK)]], rows[b], gsem[b])

        def wait_gather(b):
            pltpu.make_async_copy(
                table_sh.at[pl.ds(0, CHUNK)], rows[b], gsem[b]).wait()

        def start_out(c, b):
            pltpu.async_copy(
                rows[b], out_hbm.at[pl.ds(base + c * CHUNK, CHUNK)], osem[b])

        def wait_out(c, b):
            pltpu.make_async_copy(
                rows[b], out_hbm.at[pl.ds(base + c * CHUNK, CHUNK)], osem[b]).wait()

        for b in range(NBUF):
            start_gather(b, b)

        def group_body(g, carry):
            c0 = g * NBUF
            for b in range(NBUF):
                wait_gather(b)
                start_out(c0 + b, b)
            for b in range(NBUF):
                wait_out(c0 + b, b)
                start_gather(c0 + NBUF + b, b)
            return carry

        lax.fori_loop(0, n_groups - 1, group_body, 0)

        c0 = (n_groups - 1) * NBUF
        for b in range(NBUF):
            wait_gather(b)
            start_out(c0 + b, b)
        for b in range(NBUF):
            wait_out(c0 + b, b)

    return gather


def kernel(t, pos_embedding):
    b, h = t.shape
    n_idx = b * h
    t_hmaj = jnp.swapaxes(t, 0, 1).reshape(n_idx).astype(jnp.int32)
    out = _build_gather(n_idx, pos_embedding.shape[0])(t_hmaj, pos_embedding)
    return jnp.swapaxes(out.reshape(h, b, D_MODEL), 0, 1)


# speedup vs baseline: 6.7386x; 1.4505x over previous
import functools

import jax
import jax.numpy as jnp
from jax import lax
from jax.experimental import pallas as pl
from jax.experimental.pallas import tpu as pltpu
from jax.experimental.pallas import tpu_sc as plsc

D_MODEL = 128
CHUNK = 160
NBUF = 4


@functools.lru_cache(maxsize=None)
def _build_gather(n_idx: int, n_emb: int):
    info = plsc.get_sparse_core_info()
    nc, ns = info.num_cores, info.num_subcores
    nw = nc * ns
    assert n_idx % (nw * CHUNK * NBUF) == 0
    b_per_w = n_idx // nw
    n_groups = b_per_w // (CHUNK * NBUF)

    mesh = plsc.VectorSubcoreMesh(core_axis_name="c", subcore_axis_name="s")

    @functools.partial(
        pl.kernel,
        mesh=mesh,
        out_type=jax.ShapeDtypeStruct((n_idx, D_MODEL), jnp.float32),
        scratch_types=(
            [pltpu.VMEM((b_per_w,), jnp.int32)]
            + [pltpu.VMEM_SHARED((n_emb, D_MODEL), jnp.float32)]
            + [pltpu.VMEM((CHUNK, D_MODEL), jnp.float32) for _ in range(NBUF)]
            + [pltpu.SemaphoreType.DMA for _ in range(2 * NBUF)]
        ),
        compiler_params=pltpu.CompilerParams(use_tc_tiling_on_sc=True),
    )
    def gather(t_hbm, table_hbm, out_hbm, idx_v, table_sh, *bufs_and_sems):
        rows = bufs_and_sems[:NBUF]
        gsem = bufs_and_sems[NBUF:2 * NBUF]
        osem = bufs_and_sems[2 * NBUF:]

        wid = lax.axis_index("s") * nc + lax.axis_index("c")
        base = wid * b_per_w

        @pl.when(lax.axis_index("s") == 0)
        def _():
            pltpu.sync_copy(table_hbm, table_sh)

        pltpu.sync_copy(t_hbm.at[pl.ds(base, b_per_w)], idx_v)
        plsc.subcore_barrier()

        def start_gather(c, b):
            pltpu.async_copy(
                table_sh.at[idx_v.at[pl.ds(c * CHUNK, CHUNK)]], rows[b], gsem[b])

        def wait_gather(b):
            pltpu.make_async_copy(
                table_sh.at[pl.ds(0, CHUNK)], rows[b], gsem[b]).wait()

        def start_out(c, b):
            pltpu.async_copy(
                rows[b], out_hbm.at[pl.ds(base + c * CHUNK, CHUNK)], osem[b])

        def wait_out(c, b):
            pltpu.make_async_copy(
                rows[b], out_hbm.at[pl.ds(base + c * CHUNK, CHUNK)], osem[b]).wait()

        for b in range(NBUF):
            start_gather(b, b)

        def group_body(g, carry):
            c0 = g * NBUF
            for b in range(NBUF):
                wait_gather(b)
                start_out(c0 + b, b)
            for b in range(NBUF):
                wait_out(c0 + b, b)
                start_gather(c0 + NBUF + b, b)
            return carry

        lax.fori_loop(0, n_groups - 1, group_body, 0)

        c0 = (n_groups - 1) * NBUF
        for b in range(NBUF):
            wait_gather(b)
            start_out(c0 + b, b)
        for b in range(NBUF):
            wait_out(c0 + b, b)

    return gather


def kernel(t, pos_embedding):
    b, h = t.shape
    n_idx = b * h
    t_hmaj = jnp.swapaxes(t, 0, 1).reshape(n_idx).astype(jnp.int32)
    out = _build_gather(n_idx, pos_embedding.shape[0])(t_hmaj, pos_embedding)
    return jnp.swapaxes(out.reshape(h, b, D_MODEL), 0, 1)


# speedup vs baseline: 6.8237x; 1.0126x over previous
import functools

import jax
import jax.numpy as jnp
from jax import lax
from jax.experimental import pallas as pl
from jax.experimental.pallas import tpu as pltpu
from jax.experimental.pallas import tpu_sc as plsc

D_MODEL = 128
CHUNK = 80
NBUF = 8


@functools.lru_cache(maxsize=None)
def _build_gather(n_idx: int, n_emb: int):
    info = plsc.get_sparse_core_info()
    nc, ns = info.num_cores, info.num_subcores
    nw = nc * ns
    assert n_idx % (nw * CHUNK * NBUF) == 0
    b_per_w = n_idx // nw
    n_groups = b_per_w // (CHUNK * NBUF)

    mesh = plsc.VectorSubcoreMesh(core_axis_name="c", subcore_axis_name="s")

    @functools.partial(
        pl.kernel,
        mesh=mesh,
        out_type=jax.ShapeDtypeStruct((n_idx, D_MODEL), jnp.float32),
        scratch_types=(
            [pltpu.VMEM((b_per_w,), jnp.int32)]
            + [pltpu.VMEM_SHARED((n_emb, D_MODEL), jnp.float32)]
            + [pltpu.VMEM((CHUNK, D_MODEL), jnp.float32) for _ in range(NBUF)]
            + [pltpu.SemaphoreType.DMA for _ in range(2 * NBUF)]
        ),
        compiler_params=pltpu.CompilerParams(use_tc_tiling_on_sc=True),
    )
    def gather(t_hbm, table_hbm, out_hbm, idx_v, table_sh, *bufs_and_sems):
        rows = bufs_and_sems[:NBUF]
        gsem = bufs_and_sems[NBUF:2 * NBUF]
        osem = bufs_and_sems[2 * NBUF:]

        wid = lax.axis_index("s") * nc + lax.axis_index("c")
        base = wid * b_per_w

        @pl.when(lax.axis_index("s") == 0)
        def _():
            pltpu.sync_copy(table_hbm, table_sh)

        pltpu.sync_copy(t_hbm.at[pl.ds(base, b_per_w)], idx_v)
        plsc.subcore_barrier()

        def start_gather(c, b):
            pltpu.async_copy(
                table_sh.at[idx_v.at[pl.ds(c * CHUNK, CHUNK)]], rows[b], gsem[b])

        def wait_gather(b):
            pltpu.make_async_copy(
                table_sh.at[pl.ds(0, CHUNK)], rows[b], gsem[b]).wait()

        def start_out(c, b):
            pltpu.async_copy(
                rows[b], out_hbm.at[pl.ds(base + c * CHUNK, CHUNK)], osem[b])

        def wait_out(c, b):
            pltpu.make_async_copy(
                rows[b], out_hbm.at[pl.ds(base + c * CHUNK, CHUNK)], osem[b]).wait()

        for b in range(NBUF):
            start_gather(b, b)

        def group_body(g, carry):
            c0 = g * NBUF
            for b in range(NBUF):
                wait_gather(b)
                start_out(c0 + b, b)
            for b in range(NBUF):
                wait_out(c0 + b, b)
                start_gather(c0 + NBUF + b, b)
            return carry

        lax.fori_loop(0, n_groups - 1, group_body, 0)

        c0 = (n_groups - 1) * NBUF
        for b in range(NBUF):
            wait_gather(b)
            start_out(c0 + b, b)
        for b in range(NBUF):
            wait_out(c0 + b, b)

    return gather


def kernel(t, pos_embedding):
    b, h = t.shape
    n_idx = b * h
    t_hmaj = jnp.swapaxes(t, 0, 1).reshape(n_idx).astype(jnp.int32)
    out = _build_gather(n_idx, pos_embedding.shape[0])(t_hmaj, pos_embedding)
    return jnp.swapaxes(out.reshape(h, b, D_MODEL), 0, 1)


# speedup vs baseline: 6.8272x; 1.0005x over previous
import functools

import jax
import jax.numpy as jnp
from jax import lax
from jax.experimental import pallas as pl
from jax.experimental.pallas import tpu as pltpu
from jax.experimental.pallas import tpu_sc as plsc

D_MODEL = 128
CHUNK = 64
NBUF = 10


@functools.lru_cache(maxsize=None)
def _build_gather(n_idx: int, n_emb: int):
    info = plsc.get_sparse_core_info()
    nc, ns = info.num_cores, info.num_subcores
    nw = nc * ns
    assert n_idx % (nw * CHUNK * NBUF) == 0
    b_per_w = n_idx // nw
    n_groups = b_per_w // (CHUNK * NBUF)

    mesh = plsc.VectorSubcoreMesh(core_axis_name="c", subcore_axis_name="s")

    @functools.partial(
        pl.kernel,
        mesh=mesh,
        out_type=jax.ShapeDtypeStruct((n_idx, D_MODEL), jnp.float32),
        scratch_types=(
            [pltpu.VMEM((b_per_w,), jnp.int32)]
            + [pltpu.VMEM_SHARED((n_emb, D_MODEL), jnp.float32)]
            + [pltpu.VMEM((CHUNK, D_MODEL), jnp.float32) for _ in range(NBUF)]
            + [pltpu.SemaphoreType.DMA for _ in range(2 * NBUF)]
        ),
        compiler_params=pltpu.CompilerParams(use_tc_tiling_on_sc=True),
    )
    def gather(t_hbm, table_hbm, out_hbm, idx_v, table_sh, *bufs_and_sems):
        rows = bufs_and_sems[:NBUF]
        gsem = bufs_and_sems[NBUF:2 * NBUF]
        osem = bufs_and_sems[2 * NBUF:]

        wid = lax.axis_index("s") * nc + lax.axis_index("c")
        base = wid * b_per_w

        @pl.when(lax.axis_index("s") == 0)
        def _():
            pltpu.sync_copy(table_hbm, table_sh)

        pltpu.sync_copy(t_hbm.at[pl.ds(base, b_per_w)], idx_v)
        plsc.subcore_barrier()

        def start_gather(c, b):
            pltpu.async_copy(
                table_sh.at[idx_v.at[pl.ds(c * CHUNK, CHUNK)]], rows[b], gsem[b])

        def wait_gather(b):
            pltpu.make_async_copy(
                table_sh.at[pl.ds(0, CHUNK)], rows[b], gsem[b]).wait()

        def start_out(c, b):
            pltpu.async_copy(
                rows[b], out_hbm.at[pl.ds(base + c * CHUNK, CHUNK)], osem[b])

        def wait_out(c, b):
            pltpu.make_async_copy(
                rows[b], out_hbm.at[pl.ds(base + c * CHUNK, CHUNK)], osem[b]).wait()

        for b in range(NBUF):
            start_gather(b, b)

        def group_body(g, carry):
            c0 = g * NBUF
            for b in range(NBUF):
                wait_gather(b)
                start_out(c0 + b, b)
            for b in range(NBUF):
                wait_out(c0 + b, b)
                start_gather(c0 + NBUF + b, b)
            return carry

        lax.fori_loop(0, n_groups - 1, group_body, 0)

        c0 = (n_groups - 1) * NBUF
        for b in range(NBUF):
            wait_gather(b)
            start_out(c0 + b, b)
        for b in range(NBUF):
            wait_out(c0 + b, b)

    return gather


def kernel(t, pos_embedding):
    b, h = t.shape
    n_idx = b * h
    t_hmaj = jnp.swapaxes(t, 0, 1).reshape(n_idx).astype(jnp.int32)
    out = _build_gather(n_idx, pos_embedding.shape[0])(t_hmaj, pos_embedding)
    return jnp.swapaxes(out.reshape(h, b, D_MODEL), 0, 1)
